# baseline (speedup 1.0000x reference)
import functools

import jax
import jax.numpy as jnp
from jax import lax
from jax.experimental import pallas as pl
from jax.experimental.pallas import tpu as pltpu
from jax.experimental.pallas import tpu_sc as plsc

N = 10000
D = 128
H = 8
C = 16
FF = 512

NP = 10240
NW = 32
K = 128
E_TOT = 320000 + N
RPW = 88
NCH = RPW * NW
E_PAD = NCH * K

SUB_A = 4
G_A = RPW // SUB_A
SUB_B = 1
G_B = RPW // SUB_B

RB1 = 2048
RB5 = 1024
RPT = NP // 16

_SC_PARAMS = pltpu.CompilerParams(needs_layout_passes=False,
                                  use_tc_tiling_on_sc=False)
_GDN = lax.GatherDimensionNumbers(offset_dims=(), collapsed_slice_dims=(0,),
                                  start_index_map=(0,))


def _bcast_lane(v, lane):
    idx = jnp.full((16, 1), lane, jnp.int32)
    return lax.gather(v, idx, _GDN, slice_sizes=(1,),
                      mode=lax.GatherScatterMode.PROMISE_IN_BOUNDS)
_SC_MESH = plsc.VectorSubcoreMesh(core_axis_name="c", subcore_axis_name="s",
                                  num_cores=2, num_subcores=16)


def _dense_in_body(x_ref, w_ref, a_ref, xw_ref, ab_ref):
    xw = jnp.dot(x_ref[...], w_ref[...], preferred_element_type=jnp.float32)
    xw_ref[...] = xw
    ab_ref[...] = jnp.dot(xw, a_ref[...], preferred_element_type=jnp.float32)


def _dense_in(x_p, w_gat, a_cat):
    return pl.pallas_call(
        _dense_in_body,
        grid=(NP // RB1,),
        in_specs=[
            pl.BlockSpec((RB1, D), lambda i: (i, 0)),
            pl.BlockSpec((D, D), lambda i: (0, 0)),
            pl.BlockSpec((D, 2 * H), lambda i: (0, 0)),
        ],
        out_specs=[
            pl.BlockSpec((RB1, D), lambda i: (i, 0)),
            pl.BlockSpec((RB1, 2 * H), lambda i: (i, 0)),
        ],
        out_shape=[
            jax.ShapeDtypeStruct((NP, D), jnp.float32),
            jax.ShapeDtypeStruct((NP, 2 * H), jnp.float32),
        ],
    )(x_p, w_gat, a_cat)


def _edge_a_body(src_hbm, dst_hbm, ab_hbm, zer8_hbm,
                 s_hbm, den_hbm,
                 srcv, dstv, absrc, abdst, s3, sflat,
                 gsa, gsb, wsem, ssem, shared_den):
    cid = lax.axis_index("c")
    sid = lax.axis_index("s")
    wid = sid * 2 + cid
    pltpu.sync_copy(zer8_hbm.at[pl.ds(sid * RPT, RPT)],
                    shared_den.at[pl.ds(sid * RPT, RPT)])
    plsc.subcore_barrier()

    iota = lax.iota(jnp.int32, 16)
    c_vec = jnp.bitwise_and(iota, 7)
    r0 = jnp.right_shift(iota, 3)
    row0 = wid * RPW

    def fetch(g, b):
        pltpu.sync_copy(src_hbm.at[pl.ds(row0 + g * SUB_A, SUB_A)], srcv[b])
        pltpu.sync_copy(dst_hbm.at[pl.ds(row0 + g * SUB_A, SUB_A)], dstv[b])
        for j in range(SUB_A):
            pltpu.async_copy(ab_hbm.at[srcv[b].at[j]], absrc[b].at[j], gsa[b])
            pltpu.async_copy(ab_hbm.at[dstv[b].at[j]], abdst[b].at[j], gsb[b])

    def wait_fetch(b):
        for j in range(SUB_A):
            pltpu.make_async_copy(ab_hbm.at[srcv[b].at[j]], absrc[b].at[j],
                                  gsa[b]).wait()
            pltpu.make_async_copy(ab_hbm.at[dstv[b].at[j]], abdst[b].at[j],
                                  gsb[b]).wait()

    def drain_out(g, b):
        for j in range(SUB_A):
            pltpu.make_async_copy(sflat[b].at[j],
                                  s_hbm.at[row0 + g * SUB_A + j],
                                  wsem[b]).wait()
            pltpu.make_async_copy(s3[b].at[j],
                                  shared_den.at[dstv[b].at[j]],
                                  ssem[b]).wait()

    fetch(0, 0)

    def outer(i2, carry):
        for b in (0, 1):
            g = i2 * 2 + b
            nb = 1 - b
            if b == 0:
                @pl.when(i2 > 0)
                def _():
                    drain_out(g - 1, nb)
            else:
                drain_out(g - 1, nb)
            if b == 0:
                fetch(g + 1, nb)
            else:
                @pl.when(i2 < G_A // 2 - 1)
                def _():
                    fetch(g + 1, nb)
            wait_fetch(b)
            for j in range(SUB_A):
                abs_j = absrc[b].at[j]
                abd_j = abdst[b].at[j]
                s3_j = s3[b].at[j]

                def inner(i, c2, abs_j=abs_j, abd_j=abd_j, s3_j=s3_j, jj=j):
                    r_vec = r0 + i * 2
                    ea = (plsc.load_gather(abs_j, [r_vec, c_vec])
                          + plsc.load_gather(abd_j, [r_vec, c_vec + 8]))
                    el = jnp.where(ea >= 0.0, ea, 0.2 * ea)
                    s = jnp.exp(el)
                    plsc.store_scatter(s3_j, [r_vec, c_vec], s)
                    sflat[b][jj, pl.ds(i * 16, 16)] = s
                    return c2

                lax.fori_loop(0, K * H // 16, inner, 0)
            for j in range(SUB_A):
                pltpu.async_copy(sflat[b].at[j], s_hbm.at[row0 + g * SUB_A + j],
                                 wsem[b])
                pltpu.async_copy(s3[b].at[j], shared_den.at[dstv[b].at[j]],
                                 ssem[b], add=True)
        return carry

    lax.fori_loop(0, G_A // 2, outer, 0)
    drain_out(G_A - 1, 1)
    plsc.subcore_barrier()
    pltpu.sync_copy(shared_den.at[pl.ds(sid * RPT, RPT)],
                    den_hbm.at[pl.ds(cid * NP + sid * RPT, RPT)])


_edge_a = functools.partial(
    pl.kernel,
    out_type=[
        jax.ShapeDtypeStruct((NCH, K * H), jnp.float32),
        jax.ShapeDtypeStruct((2 * NP, H), jnp.float32),
    ],
    mesh=_SC_MESH,
    compiler_params=_SC_PARAMS,
    scratch_types=[
        [pltpu.VMEM((SUB_A, K), jnp.int32) for _ in range(2)],
        [pltpu.VMEM((SUB_A, K), jnp.int32) for _ in range(2)],
        [pltpu.VMEM((SUB_A, K, 2 * H), jnp.float32) for _ in range(2)],
        [pltpu.VMEM((SUB_A, K, 2 * H), jnp.float32) for _ in range(2)],
        [pltpu.VMEM((SUB_A, K, H), jnp.float32) for _ in range(2)],
        [pltpu.VMEM((SUB_A, K * H), jnp.float32) for _ in range(2)],
        [pltpu.SemaphoreType.DMA for _ in range(2)],
        [pltpu.SemaphoreType.DMA for _ in range(2)],
        [pltpu.SemaphoreType.DMA for _ in range(2)],
        [pltpu.SemaphoreType.DMA for _ in range(2)],
        pltpu.VMEM_SHARED((NP, H), jnp.float32),
    ],
)(_edge_a_body)


def _edge_b_body(src_hbm, dst_hbm, s_hbm, xw_hbm, zer128_hbm,
                 out_hbm,
                 srcv, dstv, sfl, xwr,
                 gsx, gss, ssem, shared_out):
    cid = lax.axis_index("c")
    sid = lax.axis_index("s")
    wid = sid * 2 + cid
    pltpu.sync_copy(zer128_hbm.at[pl.ds(sid * RPT, RPT)],
                    shared_out.at[pl.ds(sid * RPT, RPT)])
    plsc.subcore_barrier()

    iota = lax.iota(jnp.int32, 16)
    row0 = wid * RPW

    def fetch(g, b):
        pltpu.sync_copy(src_hbm.at[pl.ds(row0 + g * SUB_B, SUB_B)], srcv[b])
        pltpu.sync_copy(dst_hbm.at[pl.ds(row0 + g * SUB_B, SUB_B)], dstv[b])
        for j in range(SUB_B):
            pltpu.async_copy(xw_hbm.at[srcv[b].at[j]], xwr[b].at[j], gsx[b])
            pltpu.async_copy(s_hbm.at[row0 + g * SUB_B + j], sfl[b].at[j],
                             gss[b])

    def wait_fetch(g, b):
        for j in range(SUB_B):
            pltpu.make_async_copy(xw_hbm.at[srcv[b].at[j]], xwr[b].at[j],
                                  gsx[b]).wait()
            pltpu.make_async_copy(s_hbm.at[row0 + g * SUB_B + j],
                                  sfl[b].at[j], gss[b]).wait()

    def drain_out(b):
        for j in range(SUB_B):
            pltpu.make_async_copy(xwr[b].at[j],
                                  shared_out.at[dstv[b].at[j]],
                                  ssem[b]).wait()

    fetch(0, 0)

    def outer(i2, carry):
        for b in (0, 1):
            g = i2 * 2 + b
            nb = 1 - b
            if b == 0:
                @pl.when(i2 > 0)
                def _():
                    drain_out(nb)
            else:
                drain_out(nb)
            if b == 0:
                fetch(g + 1, nb)
            else:
                @pl.when(i2 < G_B // 2 - 1)
                def _():
                    fetch(g + 1, nb)
            wait_fetch(g, b)
            for j in range(SUB_B):
                xwr_j = xwr[b].at[j]

                def scale(kk, c2, xwr_j=xwr_j, jj=j):
                    srow = sfl[b][jj, pl.ds(kk * 16, 16)]
                    for e in (0, 1):
                        rk = jnp.full((16,), 2 * kk + e, jnp.int32)
                        for h in range(H):
                            bc = _bcast_lane(srow, e * 8 + h)
                            colh = iota + h * C
                            v = plsc.load_gather(xwr_j, [rk, colh]) * bc
                            plsc.store_scatter(xwr_j, [rk, colh], v)
                    return c2

                lax.fori_loop(0, K // 2, scale, 0)
            for j in range(SUB_B):
                pltpu.async_copy(xwr[b].at[j], shared_out.at[dstv[b].at[j]],
                                 ssem[b], add=True)
        return carry

    lax.fori_loop(0, G_B // 2, outer, 0)
    drain_out(1)
    plsc.subcore_barrier()
    pltpu.sync_copy(shared_out.at[pl.ds(sid * RPT, RPT)],
                    out_hbm.at[pl.ds(cid * NP + sid * RPT, RPT)])


_edge_b = functools.partial(
    pl.kernel,
    out_type=[jax.ShapeDtypeStruct((2 * NP, D), jnp.float32)],
    mesh=_SC_MESH,
    compiler_params=_SC_PARAMS,
    scratch_types=[
        [pltpu.VMEM((SUB_B, K), jnp.int32) for _ in range(2)],
        [pltpu.VMEM((SUB_B, K), jnp.int32) for _ in range(2)],
        [pltpu.VMEM((SUB_B, K * H), jnp.float32) for _ in range(2)],
        [pltpu.VMEM((SUB_B, K, D), jnp.float32) for _ in range(2)],
        [pltpu.SemaphoreType.DMA for _ in range(2)],
        [pltpu.SemaphoreType.DMA for _ in range(2)],
        [pltpu.SemaphoreType.DMA for _ in range(2)],
        pltpu.VMEM_SHARED((NP, D), jnp.float32),
    ],
)(_edge_b_body)


def _dense_out_body(x_ref, p0_ref, p1_ref, d0_ref, d1_ref, ex_ref,
                    bg_ref, w1_ref, b1_ref, w2_ref, b2_ref,
                    g1_ref, bt1_ref, g2_ref, bt2_ref, out_ref):
    r8 = 1.0 / (d0_ref[...] + d1_ref[...] + 1e-16)
    r128 = jnp.dot(r8, ex_ref[...], preferred_element_type=jnp.float32)
    h_gat = (p0_ref[...] + p1_ref[...]) * r128 + bg_ref[...]
    t = x_ref[...] + h_gat
    mu = jnp.mean(t, axis=-1, keepdims=True)
    var = jnp.mean((t - mu) ** 2, axis=-1, keepdims=True)
    h1 = (t - mu) * lax.rsqrt(var + 1e-5) * g1_ref[...] + bt1_ref[...]
    m = jnp.dot(h1, w1_ref[...], preferred_element_type=jnp.float32) + b1_ref[...]
    m = jnp.maximum(m, 0.0)
    hf = jnp.dot(m, w2_ref[...], preferred_element_type=jnp.float32) + b2_ref[...]
    t2 = h1 + hf
    mu2 = jnp.mean(t2, axis=-1, keepdims=True)
    var2 = jnp.mean((t2 - mu2) ** 2, axis=-1, keepdims=True)
    out_ref[...] = ((t2 - mu2) * lax.rsqrt(var2 + 1e-5) * g2_ref[...]
                    + bt2_ref[...])


def _dense_out(x_p, parts, den, expand, b_gat, w1, b1, w2, b2,
               g1, bt1, g2, bt2):
    nb = NP // RB5
    full = lambda s: pl.BlockSpec(s, lambda i: (0, 0))
    return pl.pallas_call(
        _dense_out_body,
        grid=(nb,),
        in_specs=[
            pl.BlockSpec((RB5, D), lambda i: (i, 0)),
            pl.BlockSpec((RB5, D), lambda i: (i, 0)),
            pl.BlockSpec((RB5, D), lambda i: (i + nb, 0)),
            pl.BlockSpec((RB5, H), lambda i: (i, 0)),
            pl.BlockSpec((RB5, H), lambda i: (i + nb, 0)),
            full((H, D)),
            full((1, D)), full((D, FF)), full((1, FF)),
            full((FF, D)), full((1, D)), full((1, D)),
            full((1, D)), full((1, D)), full((1, D)),
        ],
        out_specs=pl.BlockSpec((RB5, D), lambda i: (i, 0)),
        out_shape=jax.ShapeDtypeStruct((NP, D), jnp.float32),
    )(x_p, parts, parts, den, den, expand, b_gat.reshape(1, D), w1,
      b1.reshape(1, FF), w2, b2.reshape(1, D), g1.reshape(1, D),
      bt1.reshape(1, D), g2.reshape(1, D), bt2.reshape(1, D))


def kernel(x, edge_index, W_gat, att_src, att_dst, b_gat, W1, b1, W2, b2,
           g1, bt1, g2, bt2):
    n = x.shape[0]
    x_p = jnp.zeros((NP, D), jnp.float32).at[:n].set(x)

    ar = jnp.arange(n, dtype=edge_index.dtype)
    src = jnp.concatenate([edge_index[0], ar])
    dst = jnp.concatenate([edge_index[1], ar])
    pad = jnp.full((E_PAD - E_TOT,), n, src.dtype)
    src = jnp.concatenate([src, pad]).astype(jnp.int32).reshape(NCH, K)
    dst = jnp.concatenate([dst, pad]).astype(jnp.int32).reshape(NCH, K)

    eye = jnp.eye(H, dtype=jnp.float32)
    a_s = (att_src[:, :, None] * eye[:, None, :]).reshape(D, H)
    a_d = (att_dst[:, :, None] * eye[:, None, :]).reshape(D, H)
    a_cat = jnp.concatenate([a_s, a_d], axis=1)
    expand = jnp.repeat(eye, C, axis=1)

    xw, ab = _dense_in(x_p, W_gat, a_cat)
    zer8 = jnp.zeros((NP, H), jnp.float32)
    zer128 = jnp.zeros((NP, D), jnp.float32)

    s_e, den = _edge_a(src, dst, ab, zer8)
    (parts,) = _edge_b(src, dst, s_e, xw, zer128)

    out = _dense_out(x_p, parts, den, expand, b_gat, W1, b1, W2, b2,
                     g1, bt1, g2, bt2)
    return out[:n]


# speedup vs baseline: 1.9658x; 1.9658x over previous
import functools

import jax
import jax.numpy as jnp
from jax import lax
from jax.experimental import pallas as pl
from jax.experimental.pallas import tpu as pltpu
from jax.experimental.pallas import tpu_sc as plsc

N = 10000
D = 128
H = 8
C = 16
FF = 512

NP = 10240
NW = 32
K = 128
E_TOT = 320000 + N
RPW = 84
NCH = RPW * NW
E_PAD = NCH * K

SUB_A = 2
G_A = RPW // SUB_A
SUB_B = 1
G_B = RPW // SUB_B

RB1 = 2048
RB5 = 1024
RPT = NP // 16

_SC_PARAMS = pltpu.CompilerParams(needs_layout_passes=False,
                                  use_tc_tiling_on_sc=False)
_GDN = lax.GatherDimensionNumbers(offset_dims=(), collapsed_slice_dims=(0,),
                                  start_index_map=(0,))


def _bcast_lane(v, lane):
    idx = jnp.full((16, 1), lane, jnp.int32)
    return lax.gather(v, idx, _GDN, slice_sizes=(1,),
                      mode=lax.GatherScatterMode.PROMISE_IN_BOUNDS)
_SC_MESH = plsc.VectorSubcoreMesh(core_axis_name="c", subcore_axis_name="s",
                                  num_cores=2, num_subcores=16)


def _dense_in_body(x_ref, w_ref, a_ref, xw_ref, ab_ref):
    xw = jnp.dot(x_ref[...], w_ref[...], preferred_element_type=jnp.float32)
    xw_ref[...] = xw
    ab_ref[...] = jnp.dot(xw, a_ref[...], preferred_element_type=jnp.float32)


def _dense_in(x_p, w_gat, a_cat):
    return pl.pallas_call(
        _dense_in_body,
        grid=(NP // RB1,),
        in_specs=[
            pl.BlockSpec((RB1, D), lambda i: (i, 0)),
            pl.BlockSpec((D, D), lambda i: (0, 0)),
            pl.BlockSpec((D, 2 * H), lambda i: (0, 0)),
        ],
        out_specs=[
            pl.BlockSpec((RB1, D), lambda i: (i, 0)),
            pl.BlockSpec((RB1, 2 * H), lambda i: (i, 0)),
        ],
        out_shape=[
            jax.ShapeDtypeStruct((NP, D), jnp.float32),
            jax.ShapeDtypeStruct((NP, 2 * H), jnp.float32),
        ],
    )(x_p, w_gat, a_cat)


def _edge_a_body(src_hbm, dst_hbm, ab_hbm, zer8_hbm,
                 s_hbm, den_hbm,
                 srcv, dstv, absrc, abdst, s3, sflat,
                 gsa, gsb, wsem, ssem, shared_den):
    cid = lax.axis_index("c")
    sid = lax.axis_index("s")
    wid = sid * 2 + cid
    pltpu.sync_copy(zer8_hbm.at[pl.ds(sid * RPT, RPT)],
                    shared_den.at[pl.ds(sid * RPT, RPT)])
    plsc.subcore_barrier()

    iota = lax.iota(jnp.int32, 16)
    c_vec = jnp.bitwise_and(iota, 7)
    r0 = jnp.right_shift(iota, 3)
    row0 = wid * RPW

    def fetch(g, b):
        pltpu.sync_copy(src_hbm.at[pl.ds(row0 + g * SUB_A, SUB_A)], srcv[b])
        pltpu.sync_copy(dst_hbm.at[pl.ds(row0 + g * SUB_A, SUB_A)], dstv[b])
        for j in range(SUB_A):
            pltpu.async_copy(ab_hbm.at[srcv[b].at[j]], absrc[b].at[j], gsa[b])
            pltpu.async_copy(ab_hbm.at[dstv[b].at[j]], abdst[b].at[j], gsb[b])

    def wait_fetch(b):
        for j in range(SUB_A):
            pltpu.make_async_copy(ab_hbm.at[srcv[b].at[j]], absrc[b].at[j],
                                  gsa[b]).wait()
            pltpu.make_async_copy(ab_hbm.at[dstv[b].at[j]], abdst[b].at[j],
                                  gsb[b]).wait()

    def drain_out(g, b):
        for j in range(SUB_A):
            pltpu.make_async_copy(sflat[b].at[j],
                                  s_hbm.at[row0 + g * SUB_A + j],
                                  wsem[b]).wait()
            pltpu.make_async_copy(s3[b].at[j],
                                  shared_den.at[dstv[b].at[j]],
                                  ssem[b]).wait()

    fetch(0, 0)

    def outer(i2, carry):
        for b in (0, 1):
            g = i2 * 2 + b
            nb = 1 - b
            if b == 0:
                @pl.when(i2 > 0)
                def _():
                    drain_out(g - 1, nb)
            else:
                drain_out(g - 1, nb)
            if b == 0:
                fetch(g + 1, nb)
            else:
                @pl.when(i2 < G_A // 2 - 1)
                def _():
                    fetch(g + 1, nb)
            wait_fetch(b)
            for j in range(SUB_A):
                abs_j = absrc[b].at[j]
                abd_j = abdst[b].at[j]
                s3_j = s3[b].at[j]

                def inner(i, c2, abs_j=abs_j, abd_j=abd_j, s3_j=s3_j, jj=j):
                    r_vec = r0 + i * 2
                    ea = (plsc.load_gather(abs_j, [r_vec, c_vec])
                          + plsc.load_gather(abd_j, [r_vec, c_vec + 8]))
                    el = jnp.where(ea >= 0.0, ea, 0.2 * ea)
                    s = jnp.exp(el)
                    plsc.store_scatter(s3_j, [r_vec, c_vec], s)
                    sflat[b][jj, pl.ds(i * 16, 16)] = s
                    return c2

                lax.fori_loop(0, K * H // 16, inner, 0)
            for j in range(SUB_A):
                pltpu.async_copy(sflat[b].at[j], s_hbm.at[row0 + g * SUB_A + j],
                                 wsem[b])
                pltpu.async_copy(s3[b].at[j], shared_den.at[dstv[b].at[j]],
                                 ssem[b], add=True)
        return carry

    lax.fori_loop(0, G_A // 2, outer, 0)
    drain_out(G_A - 1, 1)
    plsc.subcore_barrier()
    pltpu.sync_copy(shared_den.at[pl.ds(sid * RPT, RPT)],
                    den_hbm.at[pl.ds(cid * NP + sid * RPT, RPT)])


_edge_a = functools.partial(
    pl.kernel,
    out_type=[
        jax.ShapeDtypeStruct((NCH, K * H), jnp.float32),
        jax.ShapeDtypeStruct((2 * NP, H), jnp.float32),
    ],
    mesh=_SC_MESH,
    compiler_params=_SC_PARAMS,
    scratch_types=[
        [pltpu.VMEM((SUB_A, K), jnp.int32) for _ in range(2)],
        [pltpu.VMEM((SUB_A, K), jnp.int32) for _ in range(2)],
        [pltpu.VMEM((SUB_A, K, 2 * H), jnp.float32) for _ in range(2)],
        [pltpu.VMEM((SUB_A, K, 2 * H), jnp.float32) for _ in range(2)],
        [pltpu.VMEM((SUB_A, K, H), jnp.float32) for _ in range(2)],
        [pltpu.VMEM((SUB_A, K * H), jnp.float32) for _ in range(2)],
        [pltpu.SemaphoreType.DMA for _ in range(2)],
        [pltpu.SemaphoreType.DMA for _ in range(2)],
        [pltpu.SemaphoreType.DMA for _ in range(2)],
        [pltpu.SemaphoreType.DMA for _ in range(2)],
        pltpu.VMEM_SHARED((NP, H), jnp.float32),
    ],
)(_edge_a_body)


def _edge_b_body(src_hbm, dst_hbm, s_hbm, xw_hbm, zer128_hbm,
                 out_hbm,
                 srcv, dstv, sfl, xwr,
                 gsx, gss, ssem, shared_out):
    cid = lax.axis_index("c")
    sid = lax.axis_index("s")
    wid = sid * 2 + cid
    pltpu.sync_copy(zer128_hbm.at[pl.ds(sid * RPT, RPT)],
                    shared_out.at[pl.ds(sid * RPT, RPT)])
    plsc.subcore_barrier()

    iota = lax.iota(jnp.int32, 16)
    row0 = wid * RPW

    def fetch(g, b):
        pltpu.sync_copy(src_hbm.at[pl.ds(row0 + g * SUB_B, SUB_B)], srcv[b])
        pltpu.sync_copy(dst_hbm.at[pl.ds(row0 + g * SUB_B, SUB_B)], dstv[b])
        for j in range(SUB_B):
            pltpu.async_copy(xw_hbm.at[srcv[b].at[j]], xwr[b].at[j], gsx[b])
            pltpu.async_copy(s_hbm.at[row0 + g * SUB_B + j], sfl[b].at[j],
                             gss[b])

    def wait_fetch(g, b):
        for j in range(SUB_B):
            pltpu.make_async_copy(xw_hbm.at[srcv[b].at[j]], xwr[b].at[j],
                                  gsx[b]).wait()
            pltpu.make_async_copy(s_hbm.at[row0 + g * SUB_B + j],
                                  sfl[b].at[j], gss[b]).wait()

    def drain_out(b):
        for j in range(SUB_B):
            pltpu.make_async_copy(xwr[b].at[j],
                                  shared_out.at[dstv[b].at[j]],
                                  ssem[b]).wait()

    fetch(0, 0)

    def outer(i2, carry):
        for b in (0, 1):
            g = i2 * 2 + b
            nb = 1 - b
            if b == 0:
                @pl.when(i2 > 0)
                def _():
                    drain_out(nb)
            else:
                drain_out(nb)
            if b == 0:
                fetch(g + 1, nb)
            else:
                @pl.when(i2 < G_B // 2 - 1)
                def _():
                    fetch(g + 1, nb)
            wait_fetch(g, b)
            for j in range(SUB_B):
                xwr_j = xwr[b].at[j]

                def scale(kk, c2, xwr_j=xwr_j, jj=j):
                    srow = sfl[b][jj, pl.ds(kk * 16, 16)]
                    for e in (0, 1):
                        rk = jnp.full((16,), 2 * kk + e, jnp.int32)
                        for h in range(H):
                            bc = _bcast_lane(srow, e * 8 + h)
                            colh = iota + h * C
                            v = plsc.load_gather(xwr_j, [rk, colh]) * bc
                            plsc.store_scatter(xwr_j, [rk, colh], v)
                    return c2

                lax.fori_loop(0, K // 2, scale, 0)
            for j in range(SUB_B):
                pltpu.async_copy(xwr[b].at[j], shared_out.at[dstv[b].at[j]],
                                 ssem[b], add=True)
        return carry

    lax.fori_loop(0, G_B // 2, outer, 0)
    drain_out(1)
    plsc.subcore_barrier()
    pltpu.sync_copy(shared_out.at[pl.ds(sid * RPT, RPT)],
                    out_hbm.at[pl.ds(cid * NP + sid * RPT, RPT)])


_edge_b = functools.partial(
    pl.kernel,
    out_type=[jax.ShapeDtypeStruct((2 * NP, D), jnp.float32)],
    mesh=_SC_MESH,
    compiler_params=_SC_PARAMS,
    scratch_types=[
        [pltpu.VMEM((SUB_B, K), jnp.int32) for _ in range(2)],
        [pltpu.VMEM((SUB_B, K), jnp.int32) for _ in range(2)],
        [pltpu.VMEM((SUB_B, K * H), jnp.float32) for _ in range(2)],
        [pltpu.VMEM((SUB_B, K, D), jnp.float32) for _ in range(2)],
        [pltpu.SemaphoreType.DMA for _ in range(2)],
        [pltpu.SemaphoreType.DMA for _ in range(2)],
        [pltpu.SemaphoreType.DMA for _ in range(2)],
        pltpu.VMEM_SHARED((NP, D), jnp.float32),
    ],
)(_edge_b_body)


def _dense_out_body(x_ref, p0_ref, p1_ref, d0_ref, d1_ref, ex_ref,
                    bg_ref, w1_ref, b1_ref, w2_ref, b2_ref,
                    g1_ref, bt1_ref, g2_ref, bt2_ref, out_ref):
    r8 = 1.0 / (d0_ref[...] + d1_ref[...] + 1e-16)
    r128 = jnp.dot(r8, ex_ref[...], preferred_element_type=jnp.float32)
    h_gat = (p0_ref[...] + p1_ref[...]) * r128 + bg_ref[...]
    t = x_ref[...] + h_gat
    mu = jnp.mean(t, axis=-1, keepdims=True)
    var = jnp.mean((t - mu) ** 2, axis=-1, keepdims=True)
    h1 = (t - mu) * lax.rsqrt(var + 1e-5) * g1_ref[...] + bt1_ref[...]
    m = jnp.dot(h1, w1_ref[...], preferred_element_type=jnp.float32) + b1_ref[...]
    m = jnp.maximum(m, 0.0)
    hf = jnp.dot(m, w2_ref[...], preferred_element_type=jnp.float32) + b2_ref[...]
    t2 = h1 + hf
    mu2 = jnp.mean(t2, axis=-1, keepdims=True)
    var2 = jnp.mean((t2 - mu2) ** 2, axis=-1, keepdims=True)
    out_ref[...] = ((t2 - mu2) * lax.rsqrt(var2 + 1e-5) * g2_ref[...]
                    + bt2_ref[...])


def _dense_out(x_p, parts, den, expand, b_gat, w1, b1, w2, b2,
               g1, bt1, g2, bt2):
    nb = NP // RB5
    full = lambda s: pl.BlockSpec(s, lambda i: (0, 0))
    return pl.pallas_call(
        _dense_out_body,
        grid=(nb,),
        in_specs=[
            pl.BlockSpec((RB5, D), lambda i: (i, 0)),
            pl.BlockSpec((RB5, D), lambda i: (i, 0)),
            pl.BlockSpec((RB5, D), lambda i: (i + nb, 0)),
            pl.BlockSpec((RB5, H), lambda i: (i, 0)),
            pl.BlockSpec((RB5, H), lambda i: (i + nb, 0)),
            full((H, D)),
            full((1, D)), full((D, FF)), full((1, FF)),
            full((FF, D)), full((1, D)), full((1, D)),
            full((1, D)), full((1, D)), full((1, D)),
        ],
        out_specs=pl.BlockSpec((RB5, D), lambda i: (i, 0)),
        out_shape=jax.ShapeDtypeStruct((NP, D), jnp.float32),
    )(x_p, parts, parts, den, den, expand, b_gat.reshape(1, D), w1,
      b1.reshape(1, FF), w2, b2.reshape(1, D), g1.reshape(1, D),
      bt1.reshape(1, D), g2.reshape(1, D), bt2.reshape(1, D))


def kernel(x, edge_index, W_gat, att_src, att_dst, b_gat, W1, b1, W2, b2,
           g1, bt1, g2, bt2):
    n = x.shape[0]
    x_p = jnp.zeros((NP, D), jnp.float32).at[:n].set(x)

    ar = jnp.arange(n, dtype=edge_index.dtype)
    src = jnp.concatenate([edge_index[0], ar])
    dst = jnp.concatenate([edge_index[1], ar])
    n_pad = E_PAD - E_TOT
    pad_src = jnp.full((n_pad,), n, src.dtype)
    pad_dst = (n + jnp.arange(n_pad, dtype=src.dtype) % (NP - n)).astype(src.dtype)
    src = jnp.concatenate([src, pad_src]).astype(jnp.int32).reshape(NCH, K)
    dst = jnp.concatenate([dst, pad_dst]).astype(jnp.int32).reshape(NCH, K)

    eye = jnp.eye(H, dtype=jnp.float32)
    a_s = (att_src[:, :, None] * eye[:, None, :]).reshape(D, H)
    a_d = (att_dst[:, :, None] * eye[:, None, :]).reshape(D, H)
    a_cat = jnp.concatenate([a_s, a_d], axis=1)
    expand = jnp.repeat(eye, C, axis=1)

    xw, ab = _dense_in(x_p, W_gat, a_cat)
    zer8 = jnp.zeros((NP, H), jnp.float32)
    zer128 = jnp.zeros((NP, D), jnp.float32)

    s_e, den = _edge_a(src, dst, ab, zer8)
    (parts,) = _edge_b(src, dst, s_e, xw, zer128)

    out = _dense_out(x_p, parts, den, expand, b_gat, W1, b1, W2, b2,
                     g1, bt1, g2, bt2)
    return out[:n]


# speedup vs baseline: 2.1114x; 1.0741x over previous
import functools

import jax
import jax.numpy as jnp
from jax import lax
from jax.experimental import pallas as pl
from jax.experimental.pallas import tpu as pltpu
from jax.experimental.pallas import tpu_sc as plsc

N = 10000
D = 128
H = 8
C = 16
FF = 512

NP = 10240
NW = 32
K = 128
E_TOT = 320000 + N
RPW = 84
NCH = RPW * NW
E_PAD = NCH * K

SUB_A = 2
G_A = RPW // SUB_A
SUB_B = 1
G_B = RPW // SUB_B

RB1 = 2048
RB5 = 1024
RPT = NP // 16

_SC_PARAMS = pltpu.CompilerParams(needs_layout_passes=False,
                                  use_tc_tiling_on_sc=False)
_GDN = lax.GatherDimensionNumbers(offset_dims=(), collapsed_slice_dims=(0,),
                                  start_index_map=(0,))


def _bcast_lane(v, lane):
    idx = jnp.full((16, 1), lane, jnp.int32)
    return lax.gather(v, idx, _GDN, slice_sizes=(1,),
                      mode=lax.GatherScatterMode.PROMISE_IN_BOUNDS)
_SC_MESH = plsc.VectorSubcoreMesh(core_axis_name="c", subcore_axis_name="s",
                                  num_cores=2, num_subcores=16)


def _dense_in_body(x_ref, w_ref, a_ref, xw_ref, ab_ref):
    xw = jnp.dot(x_ref[...], w_ref[...], preferred_element_type=jnp.float32)
    xw_ref[...] = xw
    ab_ref[...] = jnp.dot(xw, a_ref[...], preferred_element_type=jnp.float32)


def _dense_in(x_p, w_gat, a_cat):
    return pl.pallas_call(
        _dense_in_body,
        grid=(NP // RB1,),
        in_specs=[
            pl.BlockSpec((RB1, D), lambda i: (i, 0)),
            pl.BlockSpec((D, D), lambda i: (0, 0)),
            pl.BlockSpec((D, 2 * H), lambda i: (0, 0)),
        ],
        out_specs=[
            pl.BlockSpec((RB1, D), lambda i: (i, 0)),
            pl.BlockSpec((RB1, 2 * H), lambda i: (i, 0)),
        ],
        out_shape=[
            jax.ShapeDtypeStruct((NP, D), jnp.float32),
            jax.ShapeDtypeStruct((NP, 2 * H), jnp.float32),
        ],
    )(x_p, w_gat, a_cat)


def _edge_f_body(src_hbm, dst_hbm, ab_hbm, xw_hbm, zer8_hbm, zer128_hbm,
                 den_hbm, out_hbm,
                 srcv, dstv, absrc, abdst, s3, xwr,
                 gsa, gsb, gsx, sdn, son, sden, sout):
    cid = lax.axis_index("c")
    sid = lax.axis_index("s")
    wid = sid * 2 + cid
    pltpu.sync_copy(zer8_hbm.at[pl.ds(sid * RPT, RPT)],
                    sden.at[pl.ds(sid * RPT, RPT)])
    pltpu.sync_copy(zer128_hbm.at[pl.ds(sid * RPT, RPT)],
                    sout.at[pl.ds(sid * RPT, RPT)])
    plsc.subcore_barrier()

    iota = lax.iota(jnp.int32, 16)
    c_vec = jnp.bitwise_and(iota, 7)
    r0 = jnp.right_shift(iota, 3)
    row0 = wid * RPW

    def fetch(g, b):
        pltpu.sync_copy(src_hbm.at[pl.ds(row0 + g, 1)], srcv[b])
        pltpu.sync_copy(dst_hbm.at[pl.ds(row0 + g, 1)], dstv[b])
        pltpu.async_copy(ab_hbm.at[srcv[b].at[0]], absrc[b], gsa[b])
        pltpu.async_copy(ab_hbm.at[dstv[b].at[0]], abdst[b], gsb[b])
        pltpu.async_copy(xw_hbm.at[srcv[b].at[0]], xwr[b], gsx[b])

    def wait_fetch(b):
        pltpu.make_async_copy(ab_hbm.at[srcv[b].at[0]], absrc[b], gsa[b]).wait()
        pltpu.make_async_copy(ab_hbm.at[dstv[b].at[0]], abdst[b], gsb[b]).wait()
        pltpu.make_async_copy(xw_hbm.at[srcv[b].at[0]], xwr[b], gsx[b]).wait()

    def drain_out(b):
        pltpu.make_async_copy(s3[b], sden.at[dstv[b].at[0]], sdn[b]).wait()
        pltpu.make_async_copy(xwr[b], sout.at[dstv[b].at[0]], son[b]).wait()

    fetch(0, 0)

    def outer(i2, carry):
        for b in (0, 1):
            g = i2 * 2 + b
            nb = 1 - b
            if b == 0:
                @pl.when(i2 > 0)
                def _():
                    drain_out(nb)
            else:
                drain_out(nb)
            if b == 0:
                fetch(g + 1, nb)
            else:
                @pl.when(i2 < RPW // 2 - 1)
                def _():
                    fetch(g + 1, nb)
            wait_fetch(b)
            abs_b, abd_b, s3_b, xwr_b = absrc[b], abdst[b], s3[b], xwr[b]

            def logits(i, c2, abs_b=abs_b, abd_b=abd_b, s3_b=s3_b):
                r_vec = r0 + i * 2
                ea = (plsc.load_gather(abs_b, [r_vec, c_vec])
                      + plsc.load_gather(abd_b, [r_vec, c_vec + 8]))
                el = jnp.where(ea >= 0.0, ea, 0.2 * ea)
                plsc.store_scatter(s3_b, [r_vec, c_vec], jnp.exp(el))
                return c2

            lax.fori_loop(0, K * H // 16, logits, 0)

            def scale(kk, c2, s3_b=s3_b, xwr_b=xwr_b):
                rk = jnp.full((16,), kk, jnp.int32)
                for h in range(H):
                    bc = plsc.load_gather(
                        s3_b, [rk, jnp.full((16,), h, jnp.int32)])
                    colh = iota + h * C
                    v = plsc.load_gather(xwr_b, [rk, colh]) * bc
                    plsc.store_scatter(xwr_b, [rk, colh], v)
                return c2

            lax.fori_loop(0, K, scale, 0)
            pltpu.async_copy(s3[b], sden.at[dstv[b].at[0]], sdn[b], add=True)
            pltpu.async_copy(xwr[b], sout.at[dstv[b].at[0]], son[b], add=True)
        return carry

    lax.fori_loop(0, RPW // 2, outer, 0)
    drain_out(1)
    plsc.subcore_barrier()
    pltpu.sync_copy(sden.at[pl.ds(sid * RPT, RPT)],
                    den_hbm.at[pl.ds(cid * NP + sid * RPT, RPT)])
    pltpu.sync_copy(sout.at[pl.ds(sid * RPT, RPT)],
                    out_hbm.at[pl.ds(cid * NP + sid * RPT, RPT)])


_edge_f = functools.partial(
    pl.kernel,
    out_type=[
        jax.ShapeDtypeStruct((2 * NP, H), jnp.float32),
        jax.ShapeDtypeStruct((2 * NP, D), jnp.float32),
    ],
    mesh=_SC_MESH,
    compiler_params=_SC_PARAMS,
    scratch_types=[
        [pltpu.VMEM((1, K), jnp.int32) for _ in range(2)],
        [pltpu.VMEM((1, K), jnp.int32) for _ in range(2)],
        [pltpu.VMEM((K, 2 * H), jnp.float32) for _ in range(2)],
        [pltpu.VMEM((K, 2 * H), jnp.float32) for _ in range(2)],
        [pltpu.VMEM((K, H), jnp.float32) for _ in range(2)],
        [pltpu.VMEM((K, D), jnp.float32) for _ in range(2)],
        [pltpu.SemaphoreType.DMA for _ in range(2)],
        [pltpu.SemaphoreType.DMA for _ in range(2)],
        [pltpu.SemaphoreType.DMA for _ in range(2)],
        [pltpu.SemaphoreType.DMA for _ in range(2)],
        [pltpu.SemaphoreType.DMA for _ in range(2)],
        pltpu.VMEM_SHARED((NP, H), jnp.float32),
        pltpu.VMEM_SHARED((NP, D), jnp.float32),
    ],
)(_edge_f_body)


def _dense_out_body(x_ref, p0_ref, p1_ref, d0_ref, d1_ref, ex_ref,
                    bg_ref, w1_ref, b1_ref, w2_ref, b2_ref,
                    g1_ref, bt1_ref, g2_ref, bt2_ref, out_ref):
    r8 = 1.0 / (d0_ref[...] + d1_ref[...] + 1e-16)
    r128 = jnp.dot(r8, ex_ref[...], preferred_element_type=jnp.float32)
    h_gat = (p0_ref[...] + p1_ref[...]) * r128 + bg_ref[...]
    t = x_ref[...] + h_gat
    mu = jnp.mean(t, axis=-1, keepdims=True)
    var = jnp.mean((t - mu) ** 2, axis=-1, keepdims=True)
    h1 = (t - mu) * lax.rsqrt(var + 1e-5) * g1_ref[...] + bt1_ref[...]
    m = jnp.dot(h1, w1_ref[...], preferred_element_type=jnp.float32) + b1_ref[...]
    m = jnp.maximum(m, 0.0)
    hf = jnp.dot(m, w2_ref[...], preferred_element_type=jnp.float32) + b2_ref[...]
    t2 = h1 + hf
    mu2 = jnp.mean(t2, axis=-1, keepdims=True)
    var2 = jnp.mean((t2 - mu2) ** 2, axis=-1, keepdims=True)
    out_ref[...] = ((t2 - mu2) * lax.rsqrt(var2 + 1e-5) * g2_ref[...]
                    + bt2_ref[...])


def _dense_out(x_p, parts, den, expand, b_gat, w1, b1, w2, b2,
               g1, bt1, g2, bt2):
    nb = NP // RB5
    full = lambda s: pl.BlockSpec(s, lambda i: (0, 0))
    return pl.pallas_call(
        _dense_out_body,
        grid=(nb,),
        in_specs=[
            pl.BlockSpec((RB5, D), lambda i: (i, 0)),
            pl.BlockSpec((RB5, D), lambda i: (i, 0)),
            pl.BlockSpec((RB5, D), lambda i: (i + nb, 0)),
            pl.BlockSpec((RB5, H), lambda i: (i, 0)),
            pl.BlockSpec((RB5, H), lambda i: (i + nb, 0)),
            full((H, D)),
            full((1, D)), full((D, FF)), full((1, FF)),
            full((FF, D)), full((1, D)), full((1, D)),
            full((1, D)), full((1, D)), full((1, D)),
        ],
        out_specs=pl.BlockSpec((RB5, D), lambda i: (i, 0)),
        out_shape=jax.ShapeDtypeStruct((NP, D), jnp.float32),
    )(x_p, parts, parts, den, den, expand, b_gat.reshape(1, D), w1,
      b1.reshape(1, FF), w2, b2.reshape(1, D), g1.reshape(1, D),
      bt1.reshape(1, D), g2.reshape(1, D), bt2.reshape(1, D))


def kernel(x, edge_index, W_gat, att_src, att_dst, b_gat, W1, b1, W2, b2,
           g1, bt1, g2, bt2):
    n = x.shape[0]
    x_p = jnp.zeros((NP, D), jnp.float32).at[:n].set(x)

    ar = jnp.arange(n, dtype=edge_index.dtype)
    src = jnp.concatenate([edge_index[0], ar])
    dst = jnp.concatenate([edge_index[1], ar])
    n_pad = E_PAD - E_TOT
    pad_src = jnp.full((n_pad,), n, src.dtype)
    pad_dst = (n + jnp.arange(n_pad, dtype=src.dtype) % (NP - n)).astype(src.dtype)
    src = jnp.concatenate([src, pad_src]).astype(jnp.int32).reshape(NCH, K)
    dst = jnp.concatenate([dst, pad_dst]).astype(jnp.int32).reshape(NCH, K)

    eye = jnp.eye(H, dtype=jnp.float32)
    a_s = (att_src[:, :, None] * eye[:, None, :]).reshape(D, H)
    a_d = (att_dst[:, :, None] * eye[:, None, :]).reshape(D, H)
    a_cat = jnp.concatenate([a_s, a_d], axis=1)
    expand = jnp.repeat(eye, C, axis=1)

    xw, ab = _dense_in(x_p, W_gat, a_cat)
    zer8 = jnp.zeros((NP, H), jnp.float32)
    zer128 = jnp.zeros((NP, D), jnp.float32)

    den, parts = _edge_f(src, dst, ab, xw, zer8, zer128)

    out = _dense_out(x_p, parts, den, expand, b_gat, W1, b1, W2, b2,
                     g1, bt1, g2, bt2)
    return out[:n]


# speedup vs baseline: 2.3943x; 1.1340x over previous
import functools

import jax
import jax.numpy as jnp
from jax import lax
from jax.experimental import pallas as pl
from jax.experimental.pallas import tpu as pltpu
from jax.experimental.pallas import tpu_sc as plsc

N = 10000
D = 128
H = 8
C = 16
FF = 512

NP = 10240
NW = 32
K = 128
E_TOT = 320000
RPW = 80
NCH = RPW * NW
E_PAD = NCH * K

SUB_A = 2
G_A = RPW // SUB_A
SUB_B = 1
G_B = RPW // SUB_B

RB1 = 2048
RB5 = 1024
RPT = NP // 16

_SC_PARAMS = pltpu.CompilerParams(needs_layout_passes=False,
                                  use_tc_tiling_on_sc=False)
_GDN = lax.GatherDimensionNumbers(offset_dims=(), collapsed_slice_dims=(0,),
                                  start_index_map=(0,))


def _bcast_lane(v, lane):
    idx = jnp.full((16, 1), lane, jnp.int32)
    return lax.gather(v, idx, _GDN, slice_sizes=(1,),
                      mode=lax.GatherScatterMode.PROMISE_IN_BOUNDS)
_SC_MESH = plsc.VectorSubcoreMesh(core_axis_name="c", subcore_axis_name="s",
                                  num_cores=2, num_subcores=16)


def _dense_in_body(x_ref, w_ref, a_ref, xw_ref, ab_ref):
    xw = jnp.dot(x_ref[...], w_ref[...], preferred_element_type=jnp.float32)
    xw_ref[...] = xw
    ab_ref[...] = jnp.dot(xw, a_ref[...], preferred_element_type=jnp.float32)


def _dense_in(x_p, w_gat, a_cat):
    return pl.pallas_call(
        _dense_in_body,
        grid=(NP // RB1,),
        in_specs=[
            pl.BlockSpec((RB1, D), lambda i: (i, 0)),
            pl.BlockSpec((D, D), lambda i: (0, 0)),
            pl.BlockSpec((D, 2 * H), lambda i: (0, 0)),
        ],
        out_specs=[
            pl.BlockSpec((RB1, D), lambda i: (i, 0)),
            pl.BlockSpec((RB1, 2 * H), lambda i: (i, 0)),
        ],
        out_shape=[
            jax.ShapeDtypeStruct((NP, D), jnp.float32),
            jax.ShapeDtypeStruct((NP, 2 * H), jnp.float32),
        ],
    )(x_p, w_gat, a_cat)


def _edge_f_body(src_hbm, dst_hbm, ab_hbm, xw_hbm, zer8_hbm, zer128_hbm,
                 den_hbm, out_hbm,
                 srcv, dstv, absrc, abdst, s3, xwr,
                 gsa, gsb, gsx, sdn, son, sden, sout):
    cid = lax.axis_index("c")
    sid = lax.axis_index("s")
    wid = sid * 2 + cid
    pltpu.sync_copy(zer8_hbm.at[pl.ds(sid * RPT, RPT)],
                    sden.at[pl.ds(sid * RPT, RPT)])
    pltpu.sync_copy(zer128_hbm.at[pl.ds(sid * RPT, RPT)],
                    sout.at[pl.ds(sid * RPT, RPT)])
    plsc.subcore_barrier()

    iota = lax.iota(jnp.int32, 16)
    c_vec = jnp.bitwise_and(iota, 7)
    r0 = jnp.right_shift(iota, 3)
    row0 = wid * RPW

    def fetch(g, b):
        pltpu.sync_copy(src_hbm.at[pl.ds(row0 + g, 1)], srcv[b])
        pltpu.sync_copy(dst_hbm.at[pl.ds(row0 + g, 1)], dstv[b])
        pltpu.async_copy(ab_hbm.at[srcv[b].at[0]], absrc[b], gsa[b])
        pltpu.async_copy(ab_hbm.at[dstv[b].at[0]], abdst[b], gsb[b])
        pltpu.async_copy(xw_hbm.at[srcv[b].at[0]], xwr[b], gsx[b])

    def wait_fetch(b):
        pltpu.make_async_copy(ab_hbm.at[srcv[b].at[0]], absrc[b], gsa[b]).wait()
        pltpu.make_async_copy(ab_hbm.at[dstv[b].at[0]], abdst[b], gsb[b]).wait()
        pltpu.make_async_copy(xw_hbm.at[srcv[b].at[0]], xwr[b], gsx[b]).wait()

    def drain_out(b):
        pltpu.make_async_copy(s3[b], sden.at[dstv[b].at[0]], sdn[b]).wait()
        pltpu.make_async_copy(xwr[b], sout.at[dstv[b].at[0]], son[b]).wait()

    fetch(0, 0)

    def outer(i2, carry):
        for b in (0, 1):
            g = i2 * 2 + b
            nb = 1 - b
            if b == 0:
                @pl.when(i2 > 0)
                def _():
                    drain_out(nb)
            else:
                drain_out(nb)
            if b == 0:
                fetch(g + 1, nb)
            else:
                @pl.when(i2 < RPW // 2 - 1)
                def _():
                    fetch(g + 1, nb)
            wait_fetch(b)
            abs_b, abd_b, s3_b, xwr_b = absrc[b], abdst[b], s3[b], xwr[b]

            def logits(i, c2, abs_b=abs_b, abd_b=abd_b, s3_b=s3_b):
                r_vec = r0 + i * 2
                ea = (plsc.load_gather(abs_b, [r_vec, c_vec])
                      + plsc.load_gather(abd_b, [r_vec, c_vec + 8]))
                el = jnp.where(ea >= 0.0, ea, 0.2 * ea)
                plsc.store_scatter(s3_b, [r_vec, c_vec], jnp.exp(el))
                return c2

            lax.fori_loop(0, K * H // 16, logits, 0)

            def scale(kk, c2, s3_b=s3_b, xwr_b=xwr_b):
                rk = jnp.full((16,), kk, jnp.int32)
                for h in range(H):
                    bc = plsc.load_gather(
                        s3_b, [rk, jnp.full((16,), h, jnp.int32)])
                    colh = iota + h * C
                    v = plsc.load_gather(xwr_b, [rk, colh]) * bc
                    plsc.store_scatter(xwr_b, [rk, colh], v)
                return c2

            lax.fori_loop(0, K, scale, 0)
            pltpu.async_copy(s3[b], sden.at[dstv[b].at[0]], sdn[b], add=True)
            pltpu.async_copy(xwr[b], sout.at[dstv[b].at[0]], son[b], add=True)
        return carry

    lax.fori_loop(0, RPW // 2, outer, 0)
    drain_out(1)
    plsc.subcore_barrier()
    pltpu.sync_copy(sden.at[pl.ds(sid * RPT, RPT)],
                    den_hbm.at[pl.ds(cid * NP + sid * RPT, RPT)])
    pltpu.sync_copy(sout.at[pl.ds(sid * RPT, RPT)],
                    out_hbm.at[pl.ds(cid * NP + sid * RPT, RPT)])


_edge_f = functools.partial(
    pl.kernel,
    out_type=[
        jax.ShapeDtypeStruct((2 * NP, H), jnp.float32),
        jax.ShapeDtypeStruct((2 * NP, D), jnp.float32),
    ],
    mesh=_SC_MESH,
    compiler_params=_SC_PARAMS,
    scratch_types=[
        [pltpu.VMEM((1, K), jnp.int32) for _ in range(2)],
        [pltpu.VMEM((1, K), jnp.int32) for _ in range(2)],
        [pltpu.VMEM((K, 2 * H), jnp.float32) for _ in range(2)],
        [pltpu.VMEM((K, 2 * H), jnp.float32) for _ in range(2)],
        [pltpu.VMEM((K, H), jnp.float32) for _ in range(2)],
        [pltpu.VMEM((K, D), jnp.float32) for _ in range(2)],
        [pltpu.SemaphoreType.DMA for _ in range(2)],
        [pltpu.SemaphoreType.DMA for _ in range(2)],
        [pltpu.SemaphoreType.DMA for _ in range(2)],
        [pltpu.SemaphoreType.DMA for _ in range(2)],
        [pltpu.SemaphoreType.DMA for _ in range(2)],
        pltpu.VMEM_SHARED((NP, H), jnp.float32),
        pltpu.VMEM_SHARED((NP, D), jnp.float32),
    ],
)(_edge_f_body)


def _dense_out_body(x_ref, p0_ref, p1_ref, d0_ref, d1_ref, xw_ref, ab_ref,
                    ex_ref, bg_ref, w1_ref, b1_ref, w2_ref, b2_ref,
                    g1_ref, bt1_ref, g2_ref, bt2_ref, out_ref):
    ab = ab_ref[...]
    ea = ab[:, :H] + ab[:, H:]
    s_self = jnp.exp(jnp.where(ea >= 0.0, ea, 0.2 * ea))
    s128 = jnp.dot(s_self, ex_ref[...], preferred_element_type=jnp.float32)
    num = p0_ref[...] + p1_ref[...] + s128 * xw_ref[...]
    r8 = 1.0 / (d0_ref[...] + d1_ref[...] + s_self + 1e-16)
    r128 = jnp.dot(r8, ex_ref[...], preferred_element_type=jnp.float32)
    h_gat = num * r128 + bg_ref[...]
    t = x_ref[...] + h_gat
    mu = jnp.mean(t, axis=-1, keepdims=True)
    var = jnp.mean((t - mu) ** 2, axis=-1, keepdims=True)
    h1 = (t - mu) * lax.rsqrt(var + 1e-5) * g1_ref[...] + bt1_ref[...]
    m = jnp.dot(h1, w1_ref[...], preferred_element_type=jnp.float32) + b1_ref[...]
    m = jnp.maximum(m, 0.0)
    hf = jnp.dot(m, w2_ref[...], preferred_element_type=jnp.float32) + b2_ref[...]
    t2 = h1 + hf
    mu2 = jnp.mean(t2, axis=-1, keepdims=True)
    var2 = jnp.mean((t2 - mu2) ** 2, axis=-1, keepdims=True)
    out_ref[...] = ((t2 - mu2) * lax.rsqrt(var2 + 1e-5) * g2_ref[...]
                    + bt2_ref[...])


def _dense_out(x_p, parts, den, xw, ab, expand, b_gat, w1, b1, w2, b2,
               g1, bt1, g2, bt2):
    nb = NP // RB5
    full = lambda s: pl.BlockSpec(s, lambda i: (0, 0))
    return pl.pallas_call(
        _dense_out_body,
        grid=(nb,),
        in_specs=[
            pl.BlockSpec((RB5, D), lambda i: (i, 0)),
            pl.BlockSpec((RB5, D), lambda i: (i, 0)),
            pl.BlockSpec((RB5, D), lambda i: (i + nb, 0)),
            pl.BlockSpec((RB5, H), lambda i: (i, 0)),
            pl.BlockSpec((RB5, H), lambda i: (i + nb, 0)),
            pl.BlockSpec((RB5, D), lambda i: (i, 0)),
            pl.BlockSpec((RB5, 2 * H), lambda i: (i, 0)),
            full((H, D)),
            full((1, D)), full((D, FF)), full((1, FF)),
            full((FF, D)), full((1, D)), full((1, D)),
            full((1, D)), full((1, D)), full((1, D)),
        ],
        out_specs=pl.BlockSpec((RB5, D), lambda i: (i, 0)),
        out_shape=jax.ShapeDtypeStruct((NP, D), jnp.float32),
    )(x_p, parts, parts, den, den, xw, ab, expand, b_gat.reshape(1, D), w1,
      b1.reshape(1, FF), w2, b2.reshape(1, D), g1.reshape(1, D),
      bt1.reshape(1, D), g2.reshape(1, D), bt2.reshape(1, D))


def kernel(x, edge_index, W_gat, att_src, att_dst, b_gat, W1, b1, W2, b2,
           g1, bt1, g2, bt2):
    n = x.shape[0]
    x_p = jnp.zeros((NP, D), jnp.float32).at[:n].set(x)

    n_pad = E_PAD - E_TOT
    pad_src = jnp.full((n_pad,), n, edge_index.dtype)
    pad_dst = (n + jnp.arange(n_pad, dtype=edge_index.dtype) % (NP - n))
    src = jnp.concatenate([edge_index[0], pad_src]).astype(jnp.int32).reshape(NCH, K)
    dst = jnp.concatenate([edge_index[1], pad_dst]).astype(jnp.int32).reshape(NCH, K)

    eye = jnp.eye(H, dtype=jnp.float32)
    a_s = (att_src[:, :, None] * eye[:, None, :]).reshape(D, H)
    a_d = (att_dst[:, :, None] * eye[:, None, :]).reshape(D, H)
    a_cat = jnp.concatenate([a_s, a_d], axis=1)
    expand = jnp.repeat(eye, C, axis=1)

    xw, ab = _dense_in(x_p, W_gat, a_cat)
    zer8 = jnp.zeros((NP, H), jnp.float32)
    zer128 = jnp.zeros((NP, D), jnp.float32)

    den, parts = _edge_f(src, dst, ab, xw, zer8, zer128)

    out = _dense_out(x_p, parts, den, xw, ab, expand, b_gat, W1, b1, W2, b2,
                     g1, bt1, g2, bt2)
    return out[:n]


# speedup vs baseline: 2.4032x; 1.0037x over previous
import functools

import jax
import jax.numpy as jnp
from jax import lax
from jax.experimental import pallas as pl
from jax.experimental.pallas import tpu as pltpu
from jax.experimental.pallas import tpu_sc as plsc

N = 10000
D = 128
H = 8
C = 16
FF = 512

NP = 10240
NW = 32
K = 128
E_TOT = 320000
RPW = 80
NCH = RPW * NW
E_PAD = NCH * K

SUB_A = 2
G_A = RPW // SUB_A
SUB_B = 1
G_B = RPW // SUB_B

RB1 = 2048
RB5 = 1024
RPT = NP // 16

_SC_PARAMS = pltpu.CompilerParams(needs_layout_passes=False,
                                  use_tc_tiling_on_sc=False)
_GDN = lax.GatherDimensionNumbers(offset_dims=(), collapsed_slice_dims=(0,),
                                  start_index_map=(0,))


def _bcast_lane(v, lane):
    idx = jnp.full((16, 1), lane, jnp.int32)
    return lax.gather(v, idx, _GDN, slice_sizes=(1,),
                      mode=lax.GatherScatterMode.PROMISE_IN_BOUNDS)
_SC_MESH = plsc.VectorSubcoreMesh(core_axis_name="c", subcore_axis_name="s",
                                  num_cores=2, num_subcores=16)


def _dense_in_body(x_ref, w_ref, a_ref, xw_ref, ab_ref):
    xw = jnp.dot(x_ref[...], w_ref[...], preferred_element_type=jnp.float32)
    xw_ref[...] = xw
    ab_ref[...] = jnp.dot(xw, a_ref[...], preferred_element_type=jnp.float32)


def _dense_in(x_p, w_gat, a_cat):
    return pl.pallas_call(
        _dense_in_body,
        grid=(NP // RB1,),
        in_specs=[
            pl.BlockSpec((RB1, D), lambda i: (i, 0)),
            pl.BlockSpec((D, D), lambda i: (0, 0)),
            pl.BlockSpec((D, 2 * H), lambda i: (0, 0)),
        ],
        out_specs=[
            pl.BlockSpec((RB1, D), lambda i: (i, 0)),
            pl.BlockSpec((RB1, 2 * H), lambda i: (i, 0)),
        ],
        out_shape=[
            jax.ShapeDtypeStruct((NP, D), jnp.float32),
            jax.ShapeDtypeStruct((NP, 2 * H), jnp.float32),
        ],
    )(x_p, w_gat, a_cat)


def _edge_f_body(src_hbm, dst_hbm, ab_hbm, xw_hbm, zer8_hbm, zer128_hbm,
                 den_hbm, out_hbm,
                 srcv, dstv, absrc, abdst, s3, xwr,
                 gsa, gsb, gsx, sdn, son, sden, sout):
    cid = lax.axis_index("c")
    sid = lax.axis_index("s")
    wid = sid * 2 + cid
    pltpu.sync_copy(zer8_hbm.at[pl.ds(sid * RPT, RPT)],
                    sden.at[pl.ds(sid * RPT, RPT)])
    pltpu.sync_copy(zer128_hbm.at[pl.ds(sid * RPT, RPT)],
                    sout.at[pl.ds(sid * RPT, RPT)])
    plsc.subcore_barrier()

    iota = lax.iota(jnp.int32, 16)
    c_vec = jnp.bitwise_and(iota, 7)
    r0 = jnp.right_shift(iota, 3)
    row0 = wid * RPW

    def fetch(g, b):
        pltpu.sync_copy(src_hbm.at[pl.ds(row0 + g, 1)], srcv[b])
        pltpu.sync_copy(dst_hbm.at[pl.ds(row0 + g, 1)], dstv[b])
        pltpu.async_copy(ab_hbm.at[srcv[b].at[0]], absrc[b], gsa[b])
        pltpu.async_copy(ab_hbm.at[dstv[b].at[0]], abdst[b], gsb[b])
        pltpu.async_copy(xw_hbm.at[srcv[b].at[0]], xwr[b], gsx[b])

    def wait_fetch(b):
        pltpu.make_async_copy(ab_hbm.at[srcv[b].at[0]], absrc[b], gsa[b]).wait()
        pltpu.make_async_copy(ab_hbm.at[dstv[b].at[0]], abdst[b], gsb[b]).wait()
        pltpu.make_async_copy(xw_hbm.at[srcv[b].at[0]], xwr[b], gsx[b]).wait()

    def drain_out(b):
        pltpu.make_async_copy(s3[b], sden.at[dstv[b].at[0]], sdn[b]).wait()
        pltpu.make_async_copy(xwr[b], sout.at[dstv[b].at[0]], son[b]).wait()

    fetch(0, 0)

    def outer(i2, carry):
        for b in (0, 1):
            g = i2 * 2 + b
            nb = 1 - b
            if b == 0:
                @pl.when(i2 > 0)
                def _():
                    drain_out(nb)
            else:
                drain_out(nb)
            if b == 0:
                fetch(g + 1, nb)
            else:
                @pl.when(i2 < RPW // 2 - 1)
                def _():
                    fetch(g + 1, nb)
            wait_fetch(b)
            abs_b, abd_b, s3_b, xwr_b = absrc[b], abdst[b], s3[b], xwr[b]

            def logits(i, c2, abs_b=abs_b, abd_b=abd_b, s3_b=s3_b):
                r_vec = r0 + i * 2
                ea = (plsc.load_gather(abs_b, [r_vec, c_vec])
                      + plsc.load_gather(abd_b, [r_vec, c_vec + 8]))
                el = jnp.where(ea >= 0.0, ea, 0.2 * ea)
                plsc.store_scatter(s3_b, [r_vec, c_vec], jnp.exp(el))
                return c2

            lax.fori_loop(0, K * H // 16, logits, 0)
            pltpu.async_copy(s3[b], sden.at[dstv[b].at[0]], sdn[b], add=True)

            def scale(kk, c2, s3_b=s3_b, xwr_b=xwr_b):
                rk = jnp.full((16,), kk, jnp.int32)
                for h in range(H):
                    bc = plsc.load_gather(
                        s3_b, [rk, jnp.full((16,), h, jnp.int32)])
                    colh = iota + h * C
                    v = plsc.load_gather(xwr_b, [rk, colh]) * bc
                    plsc.store_scatter(xwr_b, [rk, colh], v)
                return c2

            lax.fori_loop(0, K, scale, 0)
            pltpu.async_copy(xwr[b], sout.at[dstv[b].at[0]], son[b], add=True)
        return carry

    lax.fori_loop(0, RPW // 2, outer, 0)
    drain_out(1)
    plsc.subcore_barrier()
    pltpu.sync_copy(sden.at[pl.ds(sid * RPT, RPT)],
                    den_hbm.at[pl.ds(cid * NP + sid * RPT, RPT)])
    pltpu.sync_copy(sout.at[pl.ds(sid * RPT, RPT)],
                    out_hbm.at[pl.ds(cid * NP + sid * RPT, RPT)])


_edge_f = functools.partial(
    pl.kernel,
    out_type=[
        jax.ShapeDtypeStruct((2 * NP, H), jnp.float32),
        jax.ShapeDtypeStruct((2 * NP, D), jnp.float32),
    ],
    mesh=_SC_MESH,
    compiler_params=_SC_PARAMS,
    scratch_types=[
        [pltpu.VMEM((1, K), jnp.int32) for _ in range(2)],
        [pltpu.VMEM((1, K), jnp.int32) for _ in range(2)],
        [pltpu.VMEM((K, 2 * H), jnp.float32) for _ in range(2)],
        [pltpu.VMEM((K, 2 * H), jnp.float32) for _ in range(2)],
        [pltpu.VMEM((K, H), jnp.float32) for _ in range(2)],
        [pltpu.VMEM((K, D), jnp.float32) for _ in range(2)],
        [pltpu.SemaphoreType.DMA for _ in range(2)],
        [pltpu.SemaphoreType.DMA for _ in range(2)],
        [pltpu.SemaphoreType.DMA for _ in range(2)],
        [pltpu.SemaphoreType.DMA for _ in range(2)],
        [pltpu.SemaphoreType.DMA for _ in range(2)],
        pltpu.VMEM_SHARED((NP, H), jnp.float32),
        pltpu.VMEM_SHARED((NP, D), jnp.float32),
    ],
)(_edge_f_body)


def _dense_out_body(x_ref, p0_ref, p1_ref, d0_ref, d1_ref, xw_ref, ab_ref,
                    ex_ref, bg_ref, w1_ref, b1_ref, w2_ref, b2_ref,
                    g1_ref, bt1_ref, g2_ref, bt2_ref, out_ref):
    ab = ab_ref[...]
    ea = ab[:, :H] + ab[:, H:]
    s_self = jnp.exp(jnp.where(ea >= 0.0, ea, 0.2 * ea))
    s128 = jnp.dot(s_self, ex_ref[...], preferred_element_type=jnp.float32)
    num = p0_ref[...] + p1_ref[...] + s128 * xw_ref[...]
    r8 = 1.0 / (d0_ref[...] + d1_ref[...] + s_self + 1e-16)
    r128 = jnp.dot(r8, ex_ref[...], preferred_element_type=jnp.float32)
    h_gat = num * r128 + bg_ref[...]
    t = x_ref[...] + h_gat
    mu = jnp.mean(t, axis=-1, keepdims=True)
    var = jnp.mean((t - mu) ** 2, axis=-1, keepdims=True)
    h1 = (t - mu) * lax.rsqrt(var + 1e-5) * g1_ref[...] + bt1_ref[...]
    m = jnp.dot(h1, w1_ref[...], preferred_element_type=jnp.float32) + b1_ref[...]
    m = jnp.maximum(m, 0.0)
    hf = jnp.dot(m, w2_ref[...], preferred_element_type=jnp.float32) + b2_ref[...]
    t2 = h1 + hf
    mu2 = jnp.mean(t2, axis=-1, keepdims=True)
    var2 = jnp.mean((t2 - mu2) ** 2, axis=-1, keepdims=True)
    out_ref[...] = ((t2 - mu2) * lax.rsqrt(var2 + 1e-5) * g2_ref[...]
                    + bt2_ref[...])


def _dense_out(x_p, parts, den, xw, ab, expand, b_gat, w1, b1, w2, b2,
               g1, bt1, g2, bt2):
    nb = NP // RB5
    full = lambda s: pl.BlockSpec(s, lambda i: (0, 0))
    return pl.pallas_call(
        _dense_out_body,
        grid=(nb,),
        in_specs=[
            pl.BlockSpec((RB5, D), lambda i: (i, 0)),
            pl.BlockSpec((RB5, D), lambda i: (i, 0)),
            pl.BlockSpec((RB5, D), lambda i: (i + nb, 0)),
            pl.BlockSpec((RB5, H), lambda i: (i, 0)),
            pl.BlockSpec((RB5, H), lambda i: (i + nb, 0)),
            pl.BlockSpec((RB5, D), lambda i: (i, 0)),
            pl.BlockSpec((RB5, 2 * H), lambda i: (i, 0)),
            full((H, D)),
            full((1, D)), full((D, FF)), full((1, FF)),
            full((FF, D)), full((1, D)), full((1, D)),
            full((1, D)), full((1, D)), full((1, D)),
        ],
        out_specs=pl.BlockSpec((RB5, D), lambda i: (i, 0)),
        out_shape=jax.ShapeDtypeStruct((NP, D), jnp.float32),
    )(x_p, parts, parts, den, den, xw, ab, expand, b_gat.reshape(1, D), w1,
      b1.reshape(1, FF), w2, b2.reshape(1, D), g1.reshape(1, D),
      bt1.reshape(1, D), g2.reshape(1, D), bt2.reshape(1, D))


def kernel(x, edge_index, W_gat, att_src, att_dst, b_gat, W1, b1, W2, b2,
           g1, bt1, g2, bt2):
    n = x.shape[0]
    x_p = jnp.zeros((NP, D), jnp.float32).at[:n].set(x)

    n_pad = E_PAD - E_TOT
    pad_src = jnp.full((n_pad,), n, edge_index.dtype)
    pad_dst = (n + jnp.arange(n_pad, dtype=edge_index.dtype) % (NP - n))
    src = jnp.concatenate([edge_index[0], pad_src]).astype(jnp.int32).reshape(NCH, K)
    dst = jnp.concatenate([edge_index[1], pad_dst]).astype(jnp.int32).reshape(NCH, K)

    eye = jnp.eye(H, dtype=jnp.float32)
    a_s = (att_src[:, :, None] * eye[:, None, :]).reshape(D, H)
    a_d = (att_dst[:, :, None] * eye[:, None, :]).reshape(D, H)
    a_cat = jnp.concatenate([a_s, a_d], axis=1)
    expand = jnp.repeat(eye, C, axis=1)

    xw, ab = _dense_in(x_p, W_gat, a_cat)
    zer8 = jnp.zeros((NP, H), jnp.float32)
    zer128 = jnp.zeros((NP, D), jnp.float32)

    den, parts = _edge_f(src, dst, ab, xw, zer8, zer128)

    out = _dense_out(x_p, parts, den, xw, ab, expand, b_gat, W1, b1, W2, b2,
                     g1, bt1, g2, bt2)
    return out[:n]
